# baseline (speedup 1.0000x reference)
import numpy as np

import jax
import jax.numpy as jnp
from jax import lax
from jax.experimental import pallas as pl
from jax.experimental.pallas import tpu as pltpu
from jax.experimental.pallas import tpu_sc as plsc

N = 10000
E = 320000
D = 128
DE = 16
BN_EPS = 1e-5

NC = 2
NS = 16
NW = NC * NS
EPT = E // NW
C = 40
NCHUNK = EPT // C
NBUF = 3
NP = 10240
RPT = NP // NS
RSUB = C

_ML = np.empty((D // 2,), np.int32)
for _w in range(D // 2):
    _ML[_w] = 32 * (_w // 16) + (_w % 16)
_MH = _ML + 16


def _round_bf16_bits(z):
    u = lax.bitcast_convert_type(z, jnp.uint32)
    u = u + jnp.uint32(0x7FFF) + ((u >> 16) & jnp.uint32(1))
    return u >> 16


GB = 8
EB = E // GB
CW = GB * (D // 2)
BLK = 2000


def _mm_pack_body(a_ref, wl_ref, wh_ref, bl_ref, bh_ref, o_ref):
    a = a_ref[...]
    zl = jnp.dot(a, wl_ref[...], preferred_element_type=jnp.float32) + bl_ref[...]
    zh = jnp.dot(a, wh_ref[...], preferred_element_type=jnp.float32) + bh_ref[...]
    packed = _round_bf16_bits(zl) | (_round_bf16_bits(zh) << 16)
    o_ref[...] = lax.bitcast_convert_type(packed, jnp.int32)


def _edge_linear(edge_attr, w_lo, w_hi, b_lo, b_hi):
    ea8 = edge_attr.reshape(EB, GB * DE)
    eye = jnp.eye(GB, dtype=jnp.float32)
    wl_big = jnp.kron(eye, w_lo)
    wh_big = jnp.kron(eye, w_hi)
    bl_big = jnp.tile(b_lo, GB).reshape(1, CW)
    bh_big = jnp.tile(b_hi, GB).reshape(1, CW)
    c8 = pl.pallas_call(
        _mm_pack_body,
        grid=(EB // BLK,),
        in_specs=[
            pl.BlockSpec((BLK, GB * DE), lambda i: (i, 0)),
            pl.BlockSpec((GB * DE, CW), lambda i: (0, 0)),
            pl.BlockSpec((GB * DE, CW), lambda i: (0, 0)),
            pl.BlockSpec((1, CW), lambda i: (0, 0)),
            pl.BlockSpec((1, CW), lambda i: (0, 0)),
        ],
        out_specs=pl.BlockSpec((BLK, CW), lambda i: (i, 0)),
        out_shape=jax.ShapeDtypeStruct((EB, CW), jnp.int32),
    )(ea8, wl_big, wh_big, bl_big, bh_big)
    return c8.reshape(E, D // 2)


def _node_table(x, w_at):
    def body(x_ref, w_ref, o_ref):
        o_ref[...] = jnp.dot(
            x_ref[...], w_ref[...], preferred_element_type=jnp.float32
        )
    return pl.pallas_call(
        body,
        out_shape=jax.ShapeDtypeStruct((N, D), jnp.float32),
    )(x, w_at)


def _combine_body(p_ref, cnt_ref, x_ref, w_ref, br_ref, g_ref, bt_ref, o_ref):
    p = p_ref[...]
    s = p[0] + p[1]
    cnt = jnp.sum(cnt_ref[...], axis=0)[:, None]
    scale = g_ref[...] * (1.0 / jnp.sqrt(1.0 + BN_EPS))
    mean = jnp.where(
        cnt > 0.0,
        s / jnp.maximum(cnt, 1.0) * scale + bt_ref[...],
        0.0,
    )
    res = (
        jnp.dot(x_ref[...], w_ref[...], preferred_element_type=jnp.float32)
        + br_ref[...]
    )
    o_ref[...] = mean + res


def _combine(partials, counts, x, w_rt, b_res, gamma, beta):
    blk = 1024
    grid = NP // blk
    return pl.pallas_call(
        _combine_body,
        grid=(grid,),
        in_specs=[
            pl.BlockSpec((NC, blk, D), lambda i: (0, i, 0)),
            pl.BlockSpec((NW, blk), lambda i: (0, i)),
            pl.BlockSpec((blk, D), lambda i: (i, 0)),
            pl.BlockSpec((D, D), lambda i: (0, 0)),
            pl.BlockSpec((1, D), lambda i: (0, 0)),
            pl.BlockSpec((1, D), lambda i: (0, 0)),
            pl.BlockSpec((1, D), lambda i: (0, 0)),
        ],
        out_specs=pl.BlockSpec((blk, D), lambda i: (i, 0)),
        out_shape=jax.ShapeDtypeStruct((N, D), jnp.float32),
    )(partials, counts, x, w_rt, b_res.reshape(1, D), gamma.reshape(1, D),
      beta.reshape(1, D))


def _sc_body(y_hbm, c_hbm, row_hbm, col_hbm, out_hbm, outcnt_hbm,
             acc_sh,
             ybuf0, ybuf1, ybuf2, cbuf0, cbuf1, cbuf2,
             rowbuf0, rowbuf1, rowbuf2, colbuf0, colbuf1, colbuf2,
             cntbuf,
             sl0, sl1, sl2, sg0, sg1, sg2, ss0, ss1, ss2):
    ybuf = (ybuf0, ybuf1, ybuf2)
    cbuf = (cbuf0, cbuf1, cbuf2)
    rowbuf = (rowbuf0, rowbuf1, rowbuf2)
    colbuf = (colbuf0, colbuf1, colbuf2)
    sem_l = (sl0, sl1, sl2)
    sem_g = (sg0, sg1, sg2)
    sem_s = (ss0, ss1, ss2)

    cid = lax.axis_index("c")
    sid = lax.axis_index("s")
    wid = sid * NC + cid

    zero16 = jnp.zeros((16,), jnp.float32)

    @pl.loop(0, RSUB)
    def _zero_rows(r):
        for k in range(D // 16):
            ybuf0[r, pl.ds(k * 16, 16)] = zero16

    base = sid * RPT
    for j in range(RPT // RSUB):
        pltpu.sync_copy(ybuf0, acc_sh.at[pl.ds(base + j * RSUB, RSUB)])

    @pl.loop(0, NP // 16)
    def _zero_cnt(r):
        cntbuf[pl.ds(r * 16, 16)] = zero16

    plsc.subcore_barrier()

    ept_base = wid * EPT
    tail_valid = lax.iota(jnp.int32, 16) >= (16 - (C % 16))

    def issue_loads(g, b):
        eb = ept_base + g * C
        pltpu.async_copy(row_hbm.at[pl.ds(eb, C)], rowbuf[b], sem_l[b])
        pltpu.async_copy(col_hbm.at[pl.ds(eb, C)], colbuf[b], sem_l[b])
        pltpu.async_copy(c_hbm.at[pl.ds(eb, C)], cbuf[b], sem_l[b])

    def wait_loads(g, b):
        eb = ept_base + g * C
        pltpu.make_async_copy(row_hbm.at[pl.ds(eb, C)], rowbuf[b], sem_l[b]).wait()
        pltpu.make_async_copy(col_hbm.at[pl.ds(eb, C)], colbuf[b], sem_l[b]).wait()
        pltpu.make_async_copy(c_hbm.at[pl.ds(eb, C)], cbuf[b], sem_l[b]).wait()

    def compute_scatter(b):
        @plsc.parallel_loop(0, C, unroll=4)
        def _rows(r):
            for g in range(D // 32):
                cv = plsc.bitcast(cbuf[b][r, pl.ds(g * 16, 16)],
                                  jnp.bfloat16)
                ca, cb = plsc.unpack(cv, format=plsc.PackFormat.INTERLEAVED)
                ya = ybuf[b][r, pl.ds(g * 32, 16)]
                yb2 = ybuf[b][r, pl.ds(g * 32 + 16, 16)]
                ybuf[b][r, pl.ds(g * 32, 16)] = jnp.maximum(ya + ca, 0.0)
                ybuf[b][r, pl.ds(g * 32 + 16, 16)] = jnp.maximum(yb2 + cb, 0.0)

        for j in range(C // 16):
            cv = colbuf[b][pl.ds(j * 16, 16)]
            occ, last = plsc.scan_count(cv)
            plsc.addupdate_scatter(
                cntbuf, [cv], occ.astype(jnp.float32), mask=last
            )
        if C % 16:
            cv = colbuf[b][pl.ds(C - 16, 16)]
            occ, last = plsc.scan_count(cv, mask=tail_valid)
            plsc.addupdate_scatter(
                cntbuf, [cv], occ.astype(jnp.float32), mask=last
            )

        pltpu.async_copy(ybuf[b], acc_sh.at[colbuf[b]], sem_s[b], add=True)

    assert (NCHUNK + 2) % NBUF == 0

    @pl.loop(0, (NCHUNK + 2) // NBUF)
    def _steady(t):
        for bb in range(NBUF):
            s = t * NBUF + bb

            @pl.when(s >= 3)
            def _():
                b = bb
                pltpu.make_async_copy(
                    ybuf[b], acc_sh.at[colbuf[b]], sem_s[b]
                ).wait()

            @pl.when(s < NCHUNK)
            def _():
                issue_loads(s, bb)

            @pl.when(jnp.logical_and(s >= 1, s <= NCHUNK))
            def _():
                b = (bb - 1) % NBUF
                wait_loads(s - 1, b)
                pltpu.async_copy(y_hbm.at[rowbuf[b]], ybuf[b], sem_g[b])

            @pl.when(s >= 2)
            def _():
                b = (bb - 2) % NBUF
                pltpu.make_async_copy(
                    y_hbm.at[rowbuf[b]], ybuf[b], sem_g[b]
                ).wait()
                compute_scatter(b)

    blast = (NCHUNK - 1) % NBUF
    pltpu.make_async_copy(
        ybuf[blast], acc_sh.at[colbuf[blast]], sem_s[blast]
    ).wait()

    pltpu.sync_copy(cntbuf, outcnt_hbm.at[wid])

    plsc.subcore_barrier()

    for j in range(RPT // RSUB):
        rb = base + j * RSUB
        pltpu.sync_copy(acc_sh.at[pl.ds(rb, RSUB)], ybuf0)
        pltpu.sync_copy(ybuf0, out_hbm.at[cid, pl.ds(rb, RSUB)])


def _sc_scatter(y, c, row, col):
    mesh = plsc.VectorSubcoreMesh(core_axis_name="c", subcore_axis_name="s")
    f = pl.kernel(
        _sc_body,
        out_type=(
            jax.ShapeDtypeStruct((NC, NP, D), jnp.float32),
            jax.ShapeDtypeStruct((NW, NP), jnp.float32),
        ),
        mesh=mesh,
        scratch_types=[
            pltpu.VMEM_SHARED((NP, D), jnp.float32),
            pltpu.VMEM((C, D), jnp.float32),
            pltpu.VMEM((C, D), jnp.float32),
            pltpu.VMEM((C, D), jnp.float32),
            pltpu.VMEM((C, D // 2), jnp.int32),
            pltpu.VMEM((C, D // 2), jnp.int32),
            pltpu.VMEM((C, D // 2), jnp.int32),
            pltpu.VMEM((C,), jnp.int32),
            pltpu.VMEM((C,), jnp.int32),
            pltpu.VMEM((C,), jnp.int32),
            pltpu.VMEM((C,), jnp.int32),
            pltpu.VMEM((C,), jnp.int32),
            pltpu.VMEM((C,), jnp.int32),
            pltpu.VMEM((NP,), jnp.float32),
            pltpu.SemaphoreType.DMA,
            pltpu.SemaphoreType.DMA,
            pltpu.SemaphoreType.DMA,
            pltpu.SemaphoreType.DMA,
            pltpu.SemaphoreType.DMA,
            pltpu.SemaphoreType.DMA,
            pltpu.SemaphoreType.DMA,
            pltpu.SemaphoreType.DMA,
            pltpu.SemaphoreType.DMA,
        ],
        compiler_params=pltpu.CompilerParams(needs_layout_passes=False),
    )
    return f(y, c, row, col)


def kernel(x, edge_index, edge_attr, W_int, b_int, gamma, beta, W_res, b_res):
    x = x.astype(jnp.float32)
    edge_attr = edge_attr.astype(jnp.float32)
    ml = jnp.asarray(_ML)
    mh = jnp.asarray(_MH)
    w_at = W_int[:, :D].T
    w_et = W_int[:, D:].T
    row = edge_index[0]
    col = edge_index[1]

    y = _node_table(x, w_at)
    c = _edge_linear(edge_attr, w_et[:, ml], w_et[:, mh],
                     b_int[ml], b_int[mh])
    partials, counts = _sc_scatter(y, c, row, col)
    return _combine(partials, counts, x, W_res.T, b_res, gamma, beta)


# speedup vs baseline: 1.3109x; 1.3109x over previous
import numpy as np

import jax
import jax.numpy as jnp
from jax import lax
from jax.experimental import pallas as pl
from jax.experimental.pallas import tpu as pltpu
from jax.experimental.pallas import tpu_sc as plsc

N = 10000
E = 320000
D = 128
DE = 16
BN_EPS = 1e-5

NC = 2
NS = 16
NW = NC * NS
C = 64
GB = 8
CF = C // GB
EB = E // GB
CW = GB * (D // 2)
NCHT = E // C
NGMAX = NCHT // NW + 1
NBUF = 3
NP = 10240
RPT = NP // NS
RSUB = C

_ML = np.empty((D // 2,), np.int32)
for _w in range(D // 2):
    _ML[_w] = 32 * (_w // 16) + (_w % 16)
_MH = _ML + 16


def _round_bf16_bits(z):
    u = lax.bitcast_convert_type(z, jnp.uint32)
    u = u + jnp.uint32(0x7FFF) + ((u >> 16) & jnp.uint32(1))
    return u >> 16


BLK = 2000


def _mm_pack_body(a_ref, wl_ref, wh_ref, bl_ref, bh_ref, o_ref):
    a = a_ref[...]
    zl = jnp.dot(a, wl_ref[...], preferred_element_type=jnp.float32) + bl_ref[...]
    zh = jnp.dot(a, wh_ref[...], preferred_element_type=jnp.float32) + bh_ref[...]
    packed = _round_bf16_bits(zl) | (_round_bf16_bits(zh) << 16)
    o_ref[...] = lax.bitcast_convert_type(packed, jnp.int32)


def _edge_linear(edge_attr, w_lo, w_hi, b_lo, b_hi):
    ea8 = edge_attr.reshape(EB, GB * DE)
    eye = jnp.eye(GB, dtype=jnp.float32)
    wl_big = jnp.kron(eye, w_lo)
    wh_big = jnp.kron(eye, w_hi)
    bl_big = jnp.tile(b_lo, GB).reshape(1, CW)
    bh_big = jnp.tile(b_hi, GB).reshape(1, CW)
    return pl.pallas_call(
        _mm_pack_body,
        grid=(EB // BLK,),
        in_specs=[
            pl.BlockSpec((BLK, GB * DE), lambda i: (i, 0)),
            pl.BlockSpec((GB * DE, CW), lambda i: (0, 0)),
            pl.BlockSpec((GB * DE, CW), lambda i: (0, 0)),
            pl.BlockSpec((1, CW), lambda i: (0, 0)),
            pl.BlockSpec((1, CW), lambda i: (0, 0)),
        ],
        out_specs=pl.BlockSpec((BLK, CW), lambda i: (i, 0)),
        out_shape=jax.ShapeDtypeStruct((EB, CW), jnp.int32),
    )(ea8, wl_big, wh_big, bl_big, bh_big)


def _node_table(x, w_at):
    def body(x_ref, w_ref, o_ref):
        o_ref[...] = jnp.dot(
            x_ref[...], w_ref[...], preferred_element_type=jnp.float32
        )
    return pl.pallas_call(
        body,
        out_shape=jax.ShapeDtypeStruct((N, D), jnp.float32),
    )(x, w_at)


def _combine_body(p_ref, cnt_ref, x_ref, w_ref, br_ref, g_ref, bt_ref, o_ref):
    p = p_ref[...]
    s = p[0] + p[1]
    cnt = jnp.sum(cnt_ref[...], axis=0)[:, None]
    scale = g_ref[...] * (1.0 / jnp.sqrt(1.0 + BN_EPS))
    mean = jnp.where(
        cnt > 0.0,
        s / jnp.maximum(cnt, 1.0) * scale + bt_ref[...],
        0.0,
    )
    res = (
        jnp.dot(x_ref[...], w_ref[...], preferred_element_type=jnp.float32)
        + br_ref[...]
    )
    o_ref[...] = mean + res


def _combine(partials, counts, x, w_rt, b_res, gamma, beta):
    blk = 1024
    grid = NP // blk
    return pl.pallas_call(
        _combine_body,
        grid=(grid,),
        in_specs=[
            pl.BlockSpec((NC, blk, D), lambda i: (0, i, 0)),
            pl.BlockSpec((NW, blk), lambda i: (0, i)),
            pl.BlockSpec((blk, D), lambda i: (i, 0)),
            pl.BlockSpec((D, D), lambda i: (0, 0)),
            pl.BlockSpec((1, D), lambda i: (0, 0)),
            pl.BlockSpec((1, D), lambda i: (0, 0)),
            pl.BlockSpec((1, D), lambda i: (0, 0)),
        ],
        out_specs=pl.BlockSpec((blk, D), lambda i: (i, 0)),
        out_shape=jax.ShapeDtypeStruct((N, D), jnp.float32),
    )(partials, counts, x, w_rt, b_res.reshape(1, D), gamma.reshape(1, D),
      beta.reshape(1, D))


def _sc_body(y_hbm, c_hbm, row_hbm, col_hbm, out_hbm, outcnt_hbm,
             acc_sh,
             ybuf0, ybuf1, ybuf2, cbuf0, cbuf1, cbuf2,
             rowbuf0, rowbuf1, rowbuf2, colbuf0, colbuf1, colbuf2,
             cntbuf,
             sl0, sl1, sl2, sg0, sg1, sg2, ss0, ss1, ss2):
    ybuf = (ybuf0, ybuf1, ybuf2)
    cbuf = (cbuf0, cbuf1, cbuf2)
    rowbuf = (rowbuf0, rowbuf1, rowbuf2)
    colbuf = (colbuf0, colbuf1, colbuf2)
    sem_l = (sl0, sl1, sl2)
    sem_g = (sg0, sg1, sg2)
    sem_s = (ss0, ss1, ss2)

    cid = lax.axis_index("c")
    sid = lax.axis_index("s")
    wid = sid * NC + cid
    ng = jnp.where(wid < NCHT % NW, NCHT // NW + 1, NCHT // NW)

    zero16 = jnp.zeros((16,), jnp.float32)

    @pl.loop(0, RSUB)
    def _zero_rows(r):
        for k in range(D // 16):
            ybuf0[r, pl.ds(k * 16, 16)] = zero16

    base = sid * RPT
    for j in range(RPT // RSUB):
        pltpu.sync_copy(ybuf0, acc_sh.at[pl.ds(base + j * RSUB, RSUB)])

    @pl.loop(0, NP // 16)
    def _zero_cnt(r):
        cntbuf[pl.ds(r * 16, 16)] = zero16

    plsc.subcore_barrier()

    def issue_loads(s, b):
        k = wid + s * NW
        eb = k * C
        pltpu.async_copy(row_hbm.at[pl.ds(eb, C)], rowbuf[b], sem_l[b])
        pltpu.async_copy(col_hbm.at[pl.ds(eb, C)], colbuf[b], sem_l[b])
        pltpu.async_copy(c_hbm.at[pl.ds(k * CF, CF)], cbuf[b], sem_l[b])

    def wait_loads(s, b):
        k = wid + s * NW
        eb = k * C
        pltpu.make_async_copy(row_hbm.at[pl.ds(eb, C)], rowbuf[b], sem_l[b]).wait()
        pltpu.make_async_copy(col_hbm.at[pl.ds(eb, C)], colbuf[b], sem_l[b]).wait()
        pltpu.make_async_copy(c_hbm.at[pl.ds(k * CF, CF)], cbuf[b], sem_l[b]).wait()

    def compute_scatter(b):
        @plsc.parallel_loop(0, CF, unroll=2)
        def _folded(fr):
            for e in range(GB):
                r = fr * GB + e
                for g in range(D // 32):
                    cv = plsc.bitcast(
                        cbuf[b][fr, pl.ds(e * (D // 2) + g * 16, 16)],
                        jnp.bfloat16,
                    )
                    ca, cb = plsc.unpack(
                        cv, format=plsc.PackFormat.INTERLEAVED
                    )
                    ya = ybuf[b][r, pl.ds(g * 32, 16)]
                    yb2 = ybuf[b][r, pl.ds(g * 32 + 16, 16)]
                    ybuf[b][r, pl.ds(g * 32, 16)] = jnp.maximum(ya + ca, 0.0)
                    ybuf[b][r, pl.ds(g * 32 + 16, 16)] = jnp.maximum(
                        yb2 + cb, 0.0
                    )

        for j in range(C // 16):
            cv = colbuf[b][pl.ds(j * 16, 16)]
            occ, last = plsc.scan_count(cv)
            plsc.addupdate_scatter(
                cntbuf, [cv], occ.astype(jnp.float32), mask=last
            )

        pltpu.async_copy(ybuf[b], acc_sh.at[colbuf[b]], sem_s[b], add=True)

    TT = (NGMAX + 2 + NBUF - 1) // NBUF + 1

    @pl.loop(0, TT)
    def _steady(t):
        for bb in range(NBUF):
            s = t * NBUF + bb

            @pl.when(jnp.logical_and(s >= 3, s <= ng + 2))
            def _():
                b = bb
                pltpu.make_async_copy(
                    ybuf[b], acc_sh.at[colbuf[b]], sem_s[b]
                ).wait()

            @pl.when(s < ng)
            def _():
                issue_loads(s, bb)

            @pl.when(jnp.logical_and(s >= 1, s <= ng))
            def _():
                b = (bb - 1) % NBUF
                wait_loads(s - 1, b)
                pltpu.async_copy(y_hbm.at[rowbuf[b]], ybuf[b], sem_g[b])

            @pl.when(jnp.logical_and(s >= 2, s <= ng + 1))
            def _():
                b = (bb - 2) % NBUF
                pltpu.make_async_copy(
                    y_hbm.at[rowbuf[b]], ybuf[b], sem_g[b]
                ).wait()
                compute_scatter(b)

    pltpu.sync_copy(cntbuf, outcnt_hbm.at[wid])

    plsc.subcore_barrier()

    for j in range(RPT // RSUB):
        rb = base + j * RSUB
        pltpu.sync_copy(acc_sh.at[pl.ds(rb, RSUB)], ybuf0)
        pltpu.sync_copy(ybuf0, out_hbm.at[cid, pl.ds(rb, RSUB)])


def _sc_scatter(y, c, row, col):
    mesh = plsc.VectorSubcoreMesh(core_axis_name="c", subcore_axis_name="s")
    f = pl.kernel(
        _sc_body,
        out_type=(
            jax.ShapeDtypeStruct((NC, NP, D), jnp.float32),
            jax.ShapeDtypeStruct((NW, NP), jnp.float32),
        ),
        mesh=mesh,
        scratch_types=[
            pltpu.VMEM_SHARED((NP, D), jnp.float32),
            pltpu.VMEM((C, D), jnp.float32),
            pltpu.VMEM((C, D), jnp.float32),
            pltpu.VMEM((C, D), jnp.float32),
            pltpu.VMEM((CF, CW), jnp.int32),
            pltpu.VMEM((CF, CW), jnp.int32),
            pltpu.VMEM((CF, CW), jnp.int32),
            pltpu.VMEM((C,), jnp.int32),
            pltpu.VMEM((C,), jnp.int32),
            pltpu.VMEM((C,), jnp.int32),
            pltpu.VMEM((C,), jnp.int32),
            pltpu.VMEM((C,), jnp.int32),
            pltpu.VMEM((C,), jnp.int32),
            pltpu.VMEM((NP,), jnp.float32),
            pltpu.SemaphoreType.DMA,
            pltpu.SemaphoreType.DMA,
            pltpu.SemaphoreType.DMA,
            pltpu.SemaphoreType.DMA,
            pltpu.SemaphoreType.DMA,
            pltpu.SemaphoreType.DMA,
            pltpu.SemaphoreType.DMA,
            pltpu.SemaphoreType.DMA,
            pltpu.SemaphoreType.DMA,
        ],
        compiler_params=pltpu.CompilerParams(needs_layout_passes=False),
    )
    return f(y, c, row, col)


def kernel(x, edge_index, edge_attr, W_int, b_int, gamma, beta, W_res, b_res):
    x = x.astype(jnp.float32)
    edge_attr = edge_attr.astype(jnp.float32)
    ml = jnp.asarray(_ML)
    mh = jnp.asarray(_MH)
    w_at = W_int[:, :D].T
    w_et = W_int[:, D:].T
    row = edge_index[0]
    col = edge_index[1]

    y = _node_table(x, w_at)
    c = _edge_linear(edge_attr, w_et[:, ml], w_et[:, mh],
                     b_int[ml], b_int[mh])
    partials, counts = _sc_scatter(y, c, row, col)
    return _combine(partials, counts, x, W_res.T, b_res, gamma, beta)


# speedup vs baseline: 1.3197x; 1.0067x over previous
import numpy as np

import jax
import jax.numpy as jnp
from jax import lax
from jax.experimental import pallas as pl
from jax.experimental.pallas import tpu as pltpu
from jax.experimental.pallas import tpu_sc as plsc

N = 10000
E = 320000
D = 128
DE = 16
BN_EPS = 1e-5

NC = 2
NS = 16
NW = NC * NS
C = 64
GB = 8
CF = C // GB
EB = E // GB
CW = GB * (D // 2)
NCHT = E // C
NGMAX = NCHT // NW + 1
NBUF = 3
NP = 10240
RPT = NP // NS
RSUB = C

_ML = np.empty((D // 2,), np.int32)
for _w in range(D // 2):
    _ML[_w] = 32 * (_w // 16) + (_w % 16)
_MH = _ML + 16


def _round_bf16_bits(z):
    u = lax.bitcast_convert_type(z, jnp.uint32)
    u = u + jnp.uint32(0x7FFF) + ((u >> 16) & jnp.uint32(1))
    return u >> 16


BLK = 2000


YBLK = 1000


def _mm_pack_body(a_ref, wl_ref, wh_ref, bl_ref, bh_ref, x_ref, wa_ref,
                  o_ref, y_ref):
    a = a_ref[...]
    zl = jnp.dot(a, wl_ref[...], preferred_element_type=jnp.float32) + bl_ref[...]
    zh = jnp.dot(a, wh_ref[...], preferred_element_type=jnp.float32) + bh_ref[...]
    packed = _round_bf16_bits(zl) | (_round_bf16_bits(zh) << 16)
    o_ref[...] = lax.bitcast_convert_type(packed, jnp.int32)
    y_ref[...] = jnp.dot(
        x_ref[...], wa_ref[...], preferred_element_type=jnp.float32
    )


def _edge_linear(edge_attr, w_lo, w_hi, b_lo, b_hi, x, w_at):
    ea8 = edge_attr.reshape(EB, GB * DE)
    eye = jnp.eye(GB, dtype=jnp.float32)
    wl_big = jnp.kron(eye, w_lo)
    wh_big = jnp.kron(eye, w_hi)
    bl_big = jnp.tile(b_lo, GB).reshape(1, CW)
    bh_big = jnp.tile(b_hi, GB).reshape(1, CW)
    return pl.pallas_call(
        _mm_pack_body,
        grid=(EB // BLK,),
        in_specs=[
            pl.BlockSpec((BLK, GB * DE), lambda i: (i, 0)),
            pl.BlockSpec((GB * DE, CW), lambda i: (0, 0)),
            pl.BlockSpec((GB * DE, CW), lambda i: (0, 0)),
            pl.BlockSpec((1, CW), lambda i: (0, 0)),
            pl.BlockSpec((1, CW), lambda i: (0, 0)),
            pl.BlockSpec((YBLK, D), lambda i: (i % 10, 0)),
            pl.BlockSpec((D, D), lambda i: (0, 0)),
        ],
        out_specs=[
            pl.BlockSpec((BLK, CW), lambda i: (i, 0)),
            pl.BlockSpec((YBLK, D), lambda i: (i % 10, 0)),
        ],
        out_shape=[
            jax.ShapeDtypeStruct((EB, CW), jnp.int32),
            jax.ShapeDtypeStruct((N, D), jnp.float32),
        ],
    )(ea8, wl_big, wh_big, bl_big, bh_big, x, w_at)


def _combine_body(p_ref, cnt_ref, x_ref, w_ref, br_ref, g_ref, bt_ref, o_ref):
    p = p_ref[...]
    s = p[0] + p[1]
    cnt = jnp.sum(cnt_ref[...], axis=0)[:, None]
    scale = g_ref[...] * (1.0 / jnp.sqrt(1.0 + BN_EPS))
    mean = jnp.where(
        cnt > 0.0,
        s / jnp.maximum(cnt, 1.0) * scale + bt_ref[...],
        0.0,
    )
    res = (
        jnp.dot(x_ref[...], w_ref[...], preferred_element_type=jnp.float32)
        + br_ref[...]
    )
    o_ref[...] = mean + res


def _combine(partials, counts, x, w_rt, b_res, gamma, beta):
    blk = 1024
    grid = NP // blk
    return pl.pallas_call(
        _combine_body,
        grid=(grid,),
        in_specs=[
            pl.BlockSpec((NC, blk, D), lambda i: (0, i, 0)),
            pl.BlockSpec((NW, blk), lambda i: (0, i)),
            pl.BlockSpec((blk, D), lambda i: (i, 0)),
            pl.BlockSpec((D, D), lambda i: (0, 0)),
            pl.BlockSpec((1, D), lambda i: (0, 0)),
            pl.BlockSpec((1, D), lambda i: (0, 0)),
            pl.BlockSpec((1, D), lambda i: (0, 0)),
        ],
        out_specs=pl.BlockSpec((blk, D), lambda i: (i, 0)),
        out_shape=jax.ShapeDtypeStruct((N, D), jnp.float32),
    )(partials, counts, x, w_rt, b_res.reshape(1, D), gamma.reshape(1, D),
      beta.reshape(1, D))


def _sc_body(y_hbm, c_hbm, row_hbm, col_hbm, out_hbm, outcnt_hbm,
             acc_sh,
             ybuf0, ybuf1, ybuf2, cbuf0, cbuf1, cbuf2,
             rowbuf0, rowbuf1, rowbuf2, colbuf0, colbuf1, colbuf2,
             cntbuf,
             sl0, sl1, sl2, sg0, sg1, sg2, ss0, ss1, ss2):
    ybuf = (ybuf0, ybuf1, ybuf2)
    cbuf = (cbuf0, cbuf1, cbuf2)
    rowbuf = (rowbuf0, rowbuf1, rowbuf2)
    colbuf = (colbuf0, colbuf1, colbuf2)
    sem_l = (sl0, sl1, sl2)
    sem_g = (sg0, sg1, sg2)
    sem_s = (ss0, ss1, ss2)

    cid = lax.axis_index("c")
    sid = lax.axis_index("s")
    wid = sid * NC + cid
    ng = jnp.where(wid < NCHT % NW, NCHT // NW + 1, NCHT // NW)

    zero16 = jnp.zeros((16,), jnp.float32)

    @pl.loop(0, RSUB)
    def _zero_rows(r):
        for k in range(D // 16):
            ybuf0[r, pl.ds(k * 16, 16)] = zero16

    base = sid * RPT
    for j in range(RPT // RSUB):
        pltpu.sync_copy(ybuf0, acc_sh.at[pl.ds(base + j * RSUB, RSUB)])

    @pl.loop(0, NP // 16)
    def _zero_cnt(r):
        cntbuf[pl.ds(r * 16, 16)] = zero16

    plsc.subcore_barrier()

    def issue_loads(s, b):
        k = wid + s * NW
        eb = k * C
        pltpu.async_copy(row_hbm.at[pl.ds(eb, C)], rowbuf[b], sem_l[b])
        pltpu.async_copy(col_hbm.at[pl.ds(eb, C)], colbuf[b], sem_l[b])
        pltpu.async_copy(c_hbm.at[pl.ds(k * CF, CF)], cbuf[b], sem_l[b])

    def wait_loads(s, b):
        k = wid + s * NW
        eb = k * C
        pltpu.make_async_copy(row_hbm.at[pl.ds(eb, C)], rowbuf[b], sem_l[b]).wait()
        pltpu.make_async_copy(col_hbm.at[pl.ds(eb, C)], colbuf[b], sem_l[b]).wait()
        pltpu.make_async_copy(c_hbm.at[pl.ds(k * CF, CF)], cbuf[b], sem_l[b]).wait()

    def compute_scatter(b):
        @plsc.parallel_loop(0, CF, unroll=2)
        def _folded(fr):
            for e in range(GB):
                r = fr * GB + e
                for g in range(D // 32):
                    cv = plsc.bitcast(
                        cbuf[b][fr, pl.ds(e * (D // 2) + g * 16, 16)],
                        jnp.bfloat16,
                    )
                    ca, cb = plsc.unpack(
                        cv, format=plsc.PackFormat.INTERLEAVED
                    )
                    ya = ybuf[b][r, pl.ds(g * 32, 16)]
                    yb2 = ybuf[b][r, pl.ds(g * 32 + 16, 16)]
                    ybuf[b][r, pl.ds(g * 32, 16)] = jnp.maximum(ya + ca, 0.0)
                    ybuf[b][r, pl.ds(g * 32 + 16, 16)] = jnp.maximum(
                        yb2 + cb, 0.0
                    )

        for j in range(C // 16):
            cv = colbuf[b][pl.ds(j * 16, 16)]
            occ, last = plsc.scan_count(cv)
            plsc.addupdate_scatter(
                cntbuf, [cv], occ.astype(jnp.float32), mask=last
            )

        pltpu.async_copy(ybuf[b], acc_sh.at[colbuf[b]], sem_s[b], add=True)

    TT = (NGMAX + 2 + NBUF - 1) // NBUF + 1

    @pl.loop(0, TT)
    def _steady(t):
        for bb in range(NBUF):
            s = t * NBUF + bb

            @pl.when(jnp.logical_and(s >= 3, s <= ng + 2))
            def _():
                b = bb
                pltpu.make_async_copy(
                    ybuf[b], acc_sh.at[colbuf[b]], sem_s[b]
                ).wait()

            @pl.when(s < ng)
            def _():
                issue_loads(s, bb)

            @pl.when(jnp.logical_and(s >= 1, s <= ng))
            def _():
                b = (bb - 1) % NBUF
                wait_loads(s - 1, b)
                pltpu.async_copy(y_hbm.at[rowbuf[b]], ybuf[b], sem_g[b])

            @pl.when(jnp.logical_and(s >= 2, s <= ng + 1))
            def _():
                b = (bb - 2) % NBUF
                pltpu.make_async_copy(
                    y_hbm.at[rowbuf[b]], ybuf[b], sem_g[b]
                ).wait()
                compute_scatter(b)

    pltpu.sync_copy(cntbuf, outcnt_hbm.at[wid])

    plsc.subcore_barrier()

    for j in range(RPT // RSUB):
        rb = base + j * RSUB
        pltpu.sync_copy(acc_sh.at[pl.ds(rb, RSUB)], ybuf0)
        pltpu.sync_copy(ybuf0, out_hbm.at[cid, pl.ds(rb, RSUB)])


def _sc_scatter(y, c, row, col):
    mesh = plsc.VectorSubcoreMesh(core_axis_name="c", subcore_axis_name="s")
    f = pl.kernel(
        _sc_body,
        out_type=(
            jax.ShapeDtypeStruct((NC, NP, D), jnp.float32),
            jax.ShapeDtypeStruct((NW, NP), jnp.float32),
        ),
        mesh=mesh,
        scratch_types=[
            pltpu.VMEM_SHARED((NP, D), jnp.float32),
            pltpu.VMEM((C, D), jnp.float32),
            pltpu.VMEM((C, D), jnp.float32),
            pltpu.VMEM((C, D), jnp.float32),
            pltpu.VMEM((CF, CW), jnp.int32),
            pltpu.VMEM((CF, CW), jnp.int32),
            pltpu.VMEM((CF, CW), jnp.int32),
            pltpu.VMEM((C,), jnp.int32),
            pltpu.VMEM((C,), jnp.int32),
            pltpu.VMEM((C,), jnp.int32),
            pltpu.VMEM((C,), jnp.int32),
            pltpu.VMEM((C,), jnp.int32),
            pltpu.VMEM((C,), jnp.int32),
            pltpu.VMEM((NP,), jnp.float32),
            pltpu.SemaphoreType.DMA,
            pltpu.SemaphoreType.DMA,
            pltpu.SemaphoreType.DMA,
            pltpu.SemaphoreType.DMA,
            pltpu.SemaphoreType.DMA,
            pltpu.SemaphoreType.DMA,
            pltpu.SemaphoreType.DMA,
            pltpu.SemaphoreType.DMA,
            pltpu.SemaphoreType.DMA,
        ],
        compiler_params=pltpu.CompilerParams(
            needs_layout_passes=False, use_tc_tiling_on_sc=True
        ),
    )
    return f(y, c, row, col)


def kernel(x, edge_index, edge_attr, W_int, b_int, gamma, beta, W_res, b_res):
    x = x.astype(jnp.float32)
    edge_attr = edge_attr.astype(jnp.float32)
    ml = jnp.asarray(_ML)
    mh = jnp.asarray(_MH)
    w_at = W_int[:, :D].T
    w_et = W_int[:, D:].T
    row = edge_index[0]
    col = edge_index[1]

    c, y = _edge_linear(edge_attr, w_et[:, ml], w_et[:, mh],
                        b_int[ml], b_int[mh], x, w_at)
    partials, counts = _sc_scatter(y, c, row, col)
    return _combine(partials, counts, x, W_res.T, b_res, gamma, beta)
